# baseline (speedup 1.0000x reference)
import functools

import jax
import jax.numpy as jnp
from jax import lax
from jax.experimental import pallas as pl
from jax.experimental.pallas import tpu as pltpu
from jax.experimental.pallas import tpu_sc as plsc

V = 1000000
D = 64
W = 5
NEG = 5
B = 16384
NCTX = 2 * W
NWN = 1 + NEG

_INFO = plsc.get_sparse_core_info()
NC = _INFO.num_cores
NS = _INFO.num_subcores
NW = NC * NS
BW = B // NW
CH = 16
NCH = BW // CH
CTX_PER_CH = CH * NCTX
WN_PER_CH = CH * NWN


def _sig_clamped(x):
    s = 1.0 / (1.0 + jnp.exp(-x))
    s = jnp.where(x > 6.0, 1.0, s)
    return jnp.where(x > -6.0, s, 0.0)


def _cbow_body(emb0_hbm, emb1_hbm, ctx_t_hbm, wn_t_hbm, lens_hbm, mask_hbm,
               out_hbm,
               ctxidx_v, wnidx_v, physctx, physwn, lens_v, mask_v,
               ctxbuf, wnbuf, lossbuf, sem):
    wid = lax.axis_index("s") * NC + lax.axis_index("c")

    pltpu.sync_copy(ctx_t_hbm.at[wid], ctxidx_v)
    pltpu.sync_copy(wn_t_hbm.at[wid], wnidx_v)
    pltpu.sync_copy(lens_hbm.at[wid], lens_v)
    pltpu.sync_copy(mask_hbm.at[wid], mask_v)

    iota = lax.iota(jnp.int32, 16)

    def prep(c, carry):
        for k in range(NCTX):
            idx = ctxidx_v[k, pl.ds(c * CH, CH)]
            phys = jnp.where(idx == V, 0, idx >> 1)
            physctx[2 * c + k // 5, pl.ds((k % 5) * CH, CH)] = phys
        for r in range(NWN):
            idx = wnidx_v[r, pl.ds(c * CH, CH)]
            physwn[c, pl.ds(r * CH, CH)] = idx >> 1
        return carry

    lax.fori_loop(0, NCH, prep, 0)

    rows_c = [iota + CH * k for k in range(NCTX)]
    rows_w = [iota + CH * r for r in range(NWN)]

    def chunk(c, loss_acc):
        pltpu.async_copy(emb0_hbm.at[physctx.at[2 * c]],
                         ctxbuf.at[pl.ds(0, 80)], sem)
        pltpu.async_copy(emb0_hbm.at[physctx.at[2 * c + 1]],
                         ctxbuf.at[pl.ds(80, 80)], sem)
        pltpu.async_copy(emb1_hbm.at[physwn.at[c]], wnbuf, sem).wait()
        pltpu.make_async_copy(emb0_hbm.at[pl.ds(0, CTX_PER_CH)], ctxbuf,
                              sem).wait()

        offc, valc, offw = [], [], []
        for k in range(NCTX):
            idx = ctxidx_v[k, pl.ds(c * CH, CH)]
            offc.append((idx & 1) * D)
            valc.append(jnp.where(idx == V, 0.0, 1.0))
        for r in range(NWN):
            idx = wnidx_v[r, pl.ds(c * CH, CH)]
            offw.append((idx & 1) * D)

        def dbody(d, accs):
            col = (d + iota) & (D - 1)
            csum = plsc.load_gather(ctxbuf, [rows_c[0], col + offc[0]]) * valc[0]
            for k in range(1, NCTX):
                v = plsc.load_gather(ctxbuf, [rows_c[k], col + offc[k]])
                csum = csum + v * valc[k]
            out = []
            for r in range(NWN):
                v = plsc.load_gather(wnbuf, [rows_w[r], col + offw[r]])
                out.append(accs[r] + csum * v)
            return tuple(out)

        zero = jnp.zeros((16,), jnp.float32)
        accs = lax.fori_loop(0, D, dbody, (zero,) * NWN)

        inv_len = 1.0 / lens_v[pl.ds(c * CH, CH)]
        pos = _sig_clamped(accs[0] * inv_len)
        loss = loss_acc + 0.5 * (1.0 - pos) * (1.0 - pos)
        for r in range(NEG):
            neg = _sig_clamped(accs[1 + r] * inv_len)
            neg = neg * mask_v[r, pl.ds(c * CH, CH)]
            loss = loss + 0.5 * neg * neg
        return loss

    loss = lax.fori_loop(0, NCH, chunk, jnp.zeros((16,), jnp.float32))
    lossbuf[...] = loss
    pltpu.sync_copy(lossbuf, out_hbm.at[wid])


@jax.jit
def kernel(emb0_weight, emb1_weight, data):
    d32 = data.astype(jnp.int32)
    ctx = d32[:, :NCTX]
    lens = d32[:, NCTX].astype(jnp.float32)
    wn = d32[:, NCTX + 1 : NCTX + 1 + NWN]
    mask = d32[:, NCTX + 1 + NWN :].astype(jnp.float32)

    emb0_r = jnp.maximum(emb0_weight[:V], -1.0).reshape(V // 2, 2 * D)
    emb1_r = emb1_weight.reshape(V // 2, 2 * D)

    ctx_t = ctx.T.reshape(NCTX, NW, BW).transpose(1, 0, 2)
    wn_t = wn.T.reshape(NWN, NW, BW).transpose(1, 0, 2)
    lens_r = lens.reshape(NW, BW)
    mask_r = mask.T.reshape(NEG, NW, BW).transpose(1, 0, 2)

    run = pl.kernel(
        _cbow_body,
        out_type=jax.ShapeDtypeStruct((NW, 16), jnp.float32),
        mesh=plsc.VectorSubcoreMesh(core_axis_name="c", subcore_axis_name="s"),
        compiler_params=pltpu.CompilerParams(
            needs_layout_passes=False, use_tc_tiling_on_sc=True
        ),
        scratch_types=[
            pltpu.VMEM((NCTX, BW), jnp.int32),
            pltpu.VMEM((NWN, BW), jnp.int32),
            pltpu.VMEM((NCH * 2, CTX_PER_CH // 2), jnp.int32),
            pltpu.VMEM((NCH, WN_PER_CH), jnp.int32),
            pltpu.VMEM((BW,), jnp.float32),
            pltpu.VMEM((NEG, BW), jnp.float32),
            pltpu.VMEM((CTX_PER_CH, 2 * D), jnp.float32),
            pltpu.VMEM((WN_PER_CH, 2 * D), jnp.float32),
            pltpu.VMEM((16,), jnp.float32),
            pltpu.SemaphoreType.DMA,
        ],
    )
    partials = run(emb0_r, emb1_r, ctx_t, wn_t, lens_r, mask_r)
    return jnp.sum(partials)


# speedup vs baseline: 1.1625x; 1.1625x over previous
import functools

import jax
import jax.numpy as jnp
from jax import lax
from jax.experimental import pallas as pl
from jax.experimental.pallas import tpu as pltpu
from jax.experimental.pallas import tpu_sc as plsc

V = 1000000
D = 64
W = 5
NEG = 5
B = 16384
NCTX = 2 * W
NWN = 1 + NEG

_INFO = plsc.get_sparse_core_info()
NC = _INFO.num_cores
NS = _INFO.num_subcores
NW = NC * NS
BW = B // NW
CH = 16
NCH = BW // CH
CTX_PER_CH = CH * NCTX
WN_PER_CH = CH * NWN

PR = V // 2
TCH = 64
NG = PR // TCH
NG_PW = NG // NW
NG_EXTRA = NG - NG_PW * NW
TAIL_P = PR - NG * TCH
TAIL_C0 = NG * 2 * TCH

_PARAMS = pltpu.CompilerParams(
    needs_layout_passes=False, use_tc_tiling_on_sc=True
)


def _sig_clamped(x):
    s = 1.0 / (1.0 + jnp.exp(-x))
    s = jnp.where(x > 6.0, 1.0, s)
    return jnp.where(x > -6.0, s, 0.0)


def _transpose_body(e0t_hbm, e1t_hbm, out0_hbm, out1_hbm,
                    strip0, strip1, obuf0, obuf1, tstrip0, tstrip1,
                    s_in0, s_in1, s_out0, s_out1):
    wid = lax.axis_index("s") * NC + lax.axis_index("c")

    iota = lax.iota(jnp.int32, 16)
    sq = [16 * q + iota for q in range(4)]
    hsq = [[64 * h + 16 * q + iota for q in range(4)] for h in range(2)]
    nfull = jnp.where(wid < NG_EXTRA, NG_PW + 1, NG_PW)

    def one_table(src, dst):
        def fire_in(j, strip, sem):
            gid = wid + NW * j
            pltpu.async_copy(src.at[:, pl.ds(gid * 2 * TCH, 2 * TCH)],
                             strip, sem)

        def wait_in(strip, sem):
            pltpu.make_async_copy(src.at[:, pl.ds(0, 2 * TCH)], strip,
                                  sem).wait()

        def fire_out(j, obuf, sem):
            gid = wid + NW * j
            pltpu.async_copy(obuf, dst.at[pl.ds(gid * TCH, TCH)], sem)

        def wait_out(obuf, sem):
            pltpu.make_async_copy(obuf, dst.at[pl.ds(0, TCH)], sem).wait()

        def compute(strip, obuf):
            def prow(p0, carry):
                t = (p0 + iota) & (TCH - 1)
                c0 = t << 1
                c1 = c0 | 1
                for h, cv in ((0, c0), (1, c1)):
                    for q in range(4):
                        v = plsc.load_gather(strip, [sq[q], cv])
                        plsc.store_scatter(obuf, [t, hsq[h][q]], v)
                return carry

            lax.fori_loop(0, TCH, prow, 0)

        def chunk(j, carry):
            even = j % 2 == 0

            @pl.when(j + 1 < nfull)
            def _():
                @pl.when(even)
                def _():
                    fire_in(j + 1, strip1, s_in1)

                @pl.when(jnp.logical_not(even))
                def _():
                    fire_in(j + 1, strip0, s_in0)

            @pl.when(even)
            def _():
                wait_in(strip0, s_in0)

                @pl.when(j >= 2)
                def _():
                    wait_out(obuf0, s_out0)

                compute(strip0, obuf0)
                fire_out(j, obuf0, s_out0)

            @pl.when(jnp.logical_not(even))
            def _():
                wait_in(strip1, s_in1)

                @pl.when(j >= 2)
                def _():
                    wait_out(obuf1, s_out1)

                compute(strip1, obuf1)
                fire_out(j, obuf1, s_out1)

            return carry

        fire_in(0, strip0, s_in0)
        lax.fori_loop(0, nfull, chunk, 0)
        wait_out(obuf1, s_out1)
        wait_out(obuf0, s_out0)

    one_table(e0t_hbm, out0_hbm)
    one_table(e1t_hbm, out1_hbm)

    def tail(src_strip, obuf, dst):
        def prow(p, carry):
            for h in range(2):
                cv = 2 * p + h + iota * 0
                for q in range(4):
                    v = plsc.load_gather(src_strip, [sq[q], cv])
                    obuf[p, pl.ds(64 * h + 16 * q, 16)] = v
            return carry

        lax.fori_loop(0, TAIL_P, prow, 0)
        pltpu.sync_copy(obuf.at[pl.ds(0, TAIL_P)],
                        dst.at[pl.ds(NG * TCH, TAIL_P)])

    @pl.when(wid == 0)
    def _():
        pltpu.sync_copy(e0t_hbm.at[:, pl.ds(TAIL_C0, 2 * TAIL_P + 1)], tstrip0)
        tail(tstrip0, obuf0, out0_hbm)

    @pl.when(wid == 1)
    def _():
        pltpu.sync_copy(e1t_hbm.at[:, pl.ds(TAIL_C0, 2 * TAIL_P)], tstrip1)
        tail(tstrip1, obuf1, out1_hbm)


def _cbow_body(emb0_hbm, emb1_hbm, ctx_t_hbm, wn_t_hbm, lens_hbm, mask_hbm,
               out_hbm,
               ctxidx_v, wnidx_v, physctx, physwn, lens_v, mask_v,
               ctxbuf, wnbuf, lossbuf, sem):
    wid = lax.axis_index("s") * NC + lax.axis_index("c")

    pltpu.sync_copy(ctx_t_hbm.at[wid], ctxidx_v)
    pltpu.sync_copy(wn_t_hbm.at[wid], wnidx_v)
    pltpu.sync_copy(lens_hbm.at[wid], lens_v)
    pltpu.sync_copy(mask_hbm.at[wid], mask_v)

    iota = lax.iota(jnp.int32, 16)

    def prep(c, carry):
        for k in range(NCTX):
            idx = ctxidx_v[k, pl.ds(c * CH, CH)]
            phys = jnp.where(idx == V, 0, idx >> 1)
            physctx[2 * c + k // 5, pl.ds((k % 5) * CH, CH)] = phys
        for r in range(NWN):
            idx = wnidx_v[r, pl.ds(c * CH, CH)]
            physwn[c, pl.ds(r * CH, CH)] = idx >> 1
        return carry

    lax.fori_loop(0, NCH, prep, 0)

    rows_c = [iota + CH * k for k in range(NCTX)]
    rows_w = [iota + CH * r for r in range(NWN)]

    def chunk(c, loss_acc):
        pltpu.async_copy(emb0_hbm.at[physctx.at[2 * c]],
                         ctxbuf.at[pl.ds(0, 80)], sem)
        pltpu.async_copy(emb0_hbm.at[physctx.at[2 * c + 1]],
                         ctxbuf.at[pl.ds(80, 80)], sem)
        pltpu.async_copy(emb1_hbm.at[physwn.at[c]], wnbuf, sem).wait()
        pltpu.make_async_copy(emb0_hbm.at[pl.ds(0, CTX_PER_CH)], ctxbuf,
                              sem).wait()

        offc, valc, offw = [], [], []
        for k in range(NCTX):
            idx = ctxidx_v[k, pl.ds(c * CH, CH)]
            offc.append((idx & 1) * D)
            valc.append(jnp.where(idx == V, 0.0, 1.0))
        for r in range(NWN):
            idx = wnidx_v[r, pl.ds(c * CH, CH)]
            offw.append((idx & 1) * D)

        def dbody(d, accs):
            col = (d + iota) & (D - 1)
            csum = plsc.load_gather(ctxbuf, [rows_c[0], col + offc[0]]) * valc[0]
            for k in range(1, NCTX):
                v = plsc.load_gather(ctxbuf, [rows_c[k], col + offc[k]])
                csum = csum + v * valc[k]
            out = []
            for r in range(NWN):
                v = plsc.load_gather(wnbuf, [rows_w[r], col + offw[r]])
                out.append(accs[r] + csum * v)
            return tuple(out)

        zero = jnp.zeros((16,), jnp.float32)
        accs = lax.fori_loop(0, D, dbody, (zero,) * NWN)

        inv_len = 1.0 / lens_v[pl.ds(c * CH, CH)]
        pos = _sig_clamped(accs[0] * inv_len)
        loss = loss_acc + 0.5 * (1.0 - pos) * (1.0 - pos)
        for r in range(NEG):
            neg = _sig_clamped(accs[1 + r] * inv_len)
            neg = neg * mask_v[r, pl.ds(c * CH, CH)]
            loss = loss + 0.5 * neg * neg
        return loss

    loss = lax.fori_loop(0, NCH, chunk, jnp.zeros((16,), jnp.float32))
    lossbuf[...] = loss
    pltpu.sync_copy(lossbuf, out_hbm.at[wid])


@jax.jit
def kernel(emb0_weight, emb1_weight, data):
    d32 = data.astype(jnp.int32)
    ctx = d32[:, :NCTX]
    lens = d32[:, NCTX].astype(jnp.float32)
    wn = d32[:, NCTX + 1 : NCTX + 1 + NWN]
    mask = d32[:, NCTX + 1 + NWN :].astype(jnp.float32)

    e0t = emb0_weight.T
    e1t = emb1_weight.T

    mesh = plsc.VectorSubcoreMesh(core_axis_name="c", subcore_axis_name="s")

    run_t = pl.kernel(
        _transpose_body,
        out_type=(
            jax.ShapeDtypeStruct((PR, 2 * D), jnp.float32),
            jax.ShapeDtypeStruct((PR, 2 * D), jnp.float32),
        ),
        mesh=mesh,
        compiler_params=_PARAMS,
        scratch_types=[
            pltpu.VMEM((D, 2 * TCH), jnp.float32),
            pltpu.VMEM((D, 2 * TCH), jnp.float32),
            pltpu.VMEM((TCH, 2 * D), jnp.float32),
            pltpu.VMEM((TCH, 2 * D), jnp.float32),
            pltpu.VMEM((D, 2 * TAIL_P + 1), jnp.float32),
            pltpu.VMEM((D, 2 * TAIL_P), jnp.float32),
            pltpu.SemaphoreType.DMA,
            pltpu.SemaphoreType.DMA,
            pltpu.SemaphoreType.DMA,
            pltpu.SemaphoreType.DMA,
        ],
    )
    emb0_r, emb1_r = run_t(e0t, e1t)

    ctx_t = ctx.T.reshape(NCTX, NW, BW).transpose(1, 0, 2)
    wn_t = wn.T.reshape(NWN, NW, BW).transpose(1, 0, 2)
    lens_r = lens.reshape(NW, BW)
    mask_r = mask.T.reshape(NEG, NW, BW).transpose(1, 0, 2)

    run = pl.kernel(
        _cbow_body,
        out_type=jax.ShapeDtypeStruct((NW, 16), jnp.float32),
        mesh=mesh,
        compiler_params=_PARAMS,
        scratch_types=[
            pltpu.VMEM((NCTX, BW), jnp.int32),
            pltpu.VMEM((NWN, BW), jnp.int32),
            pltpu.VMEM((NCH * 2, CTX_PER_CH // 2), jnp.int32),
            pltpu.VMEM((NCH, WN_PER_CH), jnp.int32),
            pltpu.VMEM((BW,), jnp.float32),
            pltpu.VMEM((NEG, BW), jnp.float32),
            pltpu.VMEM((CTX_PER_CH, 2 * D), jnp.float32),
            pltpu.VMEM((WN_PER_CH, 2 * D), jnp.float32),
            pltpu.VMEM((16,), jnp.float32),
            pltpu.SemaphoreType.DMA,
        ],
    )
    partials = run(emb0_r, emb1_r, ctx_t, wn_t, lens_r, mask_r)
    return jnp.sum(partials)


# speedup vs baseline: 2.2779x; 1.9595x over previous
import functools

import jax
import jax.numpy as jnp
from jax import lax
from jax.experimental import pallas as pl
from jax.experimental.pallas import tpu as pltpu
from jax.experimental.pallas import tpu_sc as plsc

V = 1000000
D = 64
W = 5
NEG = 5
B = 16384
NCTX = 2 * W
NWN = 1 + NEG

_INFO = plsc.get_sparse_core_info()
NC = _INFO.num_cores
NS = _INFO.num_subcores
NW = NC * NS
BW = B // NW
CH = 16
NCH = BW // CH
CTX_PER_CH = CH * NCTX
WN_PER_CH = CH * NWN

PR = V // 2
TCH = 64
NG = PR // TCH
NG_PW = NG // NW
NG_EXTRA = NG - NG_PW * NW
TAIL_P = PR - NG * TCH
TAIL_C0 = NG * 2 * TCH

_PARAMS = pltpu.CompilerParams(
    needs_layout_passes=False, use_tc_tiling_on_sc=True
)


def _sig_clamped(x):
    s = 1.0 / (1.0 + jnp.exp(-x))
    s = jnp.where(x > 6.0, 1.0, s)
    return jnp.where(x > -6.0, s, 0.0)


def _transpose_body(e0t_hbm, e1t_hbm, out0_hbm, out1_hbm,
                    strip0, strip1, obuf0, obuf1, tstrip0, tstrip1,
                    s_in0, s_in1, s_out0, s_out1):
    wid = lax.axis_index("s") * NC + lax.axis_index("c")

    iota = lax.iota(jnp.int32, 16)
    sq = [16 * q + iota for q in range(4)]
    hsq = [[64 * h + 16 * q + iota for q in range(4)] for h in range(2)]
    nfull = jnp.where(wid < NG_EXTRA, NG_PW + 1, NG_PW)

    def one_table(src, dst):
        def fire_in(j, strip, sem):
            gid = wid + NW * j
            pltpu.async_copy(src.at[:, pl.ds(gid * 2 * TCH, 2 * TCH)],
                             strip, sem)

        def wait_in(strip, sem):
            pltpu.make_async_copy(src.at[:, pl.ds(0, 2 * TCH)], strip,
                                  sem).wait()

        def fire_out(j, obuf, sem):
            gid = wid + NW * j
            pltpu.async_copy(obuf, dst.at[pl.ds(gid * TCH, TCH)], sem)

        def wait_out(obuf, sem):
            pltpu.make_async_copy(obuf, dst.at[pl.ds(0, TCH)], sem).wait()

        def compute(strip, obuf):
            def prow(u, carry):
                vals, tees = [], []
                for pp in range(4):
                    p0 = u * 4 + pp
                    t = (p0 + iota) & (TCH - 1)
                    c0 = t << 1
                    for h in range(2):
                        cv = c0 | h
                        for q in range(4):
                            vals.append(
                                plsc.load_gather(strip, [sq[q], cv]))
                            tees.append((t, hsq[h][q]))
                for v, (t, cols) in zip(vals, tees):
                    plsc.store_scatter(obuf, [t, cols], v)
                return carry

            lax.fori_loop(0, TCH // 4, prow, 0)

        def chunk(j, carry):
            even = j % 2 == 0

            @pl.when(j + 1 < nfull)
            def _():
                @pl.when(even)
                def _():
                    fire_in(j + 1, strip1, s_in1)

                @pl.when(jnp.logical_not(even))
                def _():
                    fire_in(j + 1, strip0, s_in0)

            @pl.when(even)
            def _():
                wait_in(strip0, s_in0)

                @pl.when(j >= 2)
                def _():
                    wait_out(obuf0, s_out0)

                compute(strip0, obuf0)
                fire_out(j, obuf0, s_out0)

            @pl.when(jnp.logical_not(even))
            def _():
                wait_in(strip1, s_in1)

                @pl.when(j >= 2)
                def _():
                    wait_out(obuf1, s_out1)

                compute(strip1, obuf1)
                fire_out(j, obuf1, s_out1)

            return carry

        fire_in(0, strip0, s_in0)
        lax.fori_loop(0, nfull, chunk, 0)
        wait_out(obuf1, s_out1)
        wait_out(obuf0, s_out0)

    one_table(e0t_hbm, out0_hbm)
    one_table(e1t_hbm, out1_hbm)

    def tail(src_strip, obuf, dst):
        def prow(p, carry):
            for h in range(2):
                cv = 2 * p + h + iota * 0
                for q in range(4):
                    v = plsc.load_gather(src_strip, [sq[q], cv])
                    obuf[p, pl.ds(64 * h + 16 * q, 16)] = v
            return carry

        lax.fori_loop(0, TAIL_P, prow, 0)
        pltpu.sync_copy(obuf.at[pl.ds(0, TAIL_P)],
                        dst.at[pl.ds(NG * TCH, TAIL_P)])

    @pl.when(wid == 0)
    def _():
        pltpu.sync_copy(e0t_hbm.at[:, pl.ds(TAIL_C0, 2 * TAIL_P + 1)], tstrip0)
        tail(tstrip0, obuf0, out0_hbm)

    @pl.when(wid == 1)
    def _():
        pltpu.sync_copy(e1t_hbm.at[:, pl.ds(TAIL_C0, 2 * TAIL_P)], tstrip1)
        tail(tstrip1, obuf1, out1_hbm)


def _cbow_body(emb0_hbm, emb1_hbm, ctx_t_hbm, wn_t_hbm, lens_hbm, mask_hbm,
               out_hbm,
               ctxidx_v, wnidx_v, physctx, physwn, lens_v, mask_v,
               ctxbuf, wnbuf, lossbuf, sem):
    wid = lax.axis_index("s") * NC + lax.axis_index("c")

    pltpu.sync_copy(ctx_t_hbm.at[wid], ctxidx_v)
    pltpu.sync_copy(wn_t_hbm.at[wid], wnidx_v)
    pltpu.sync_copy(lens_hbm.at[wid], lens_v)
    pltpu.sync_copy(mask_hbm.at[wid], mask_v)

    iota = lax.iota(jnp.int32, 16)

    def prep(c, carry):
        for k in range(NCTX):
            idx = ctxidx_v[k, pl.ds(c * CH, CH)]
            phys = jnp.where(idx == V, 0, idx >> 1)
            physctx[2 * c + k // 5, pl.ds((k % 5) * CH, CH)] = phys
        for r in range(NWN):
            idx = wnidx_v[r, pl.ds(c * CH, CH)]
            physwn[c, pl.ds(r * CH, CH)] = idx >> 1
        return carry

    lax.fori_loop(0, NCH, prep, 0)

    rows_c = [iota + CH * k for k in range(NCTX)]
    rows_w = [iota + CH * r for r in range(NWN)]

    def chunk(c, loss_acc):
        pltpu.async_copy(emb0_hbm.at[physctx.at[2 * c]],
                         ctxbuf.at[pl.ds(0, 80)], sem)
        pltpu.async_copy(emb0_hbm.at[physctx.at[2 * c + 1]],
                         ctxbuf.at[pl.ds(80, 80)], sem)
        pltpu.async_copy(emb1_hbm.at[physwn.at[c]], wnbuf, sem).wait()
        pltpu.make_async_copy(emb0_hbm.at[pl.ds(0, CTX_PER_CH)], ctxbuf,
                              sem).wait()

        offc, valc, offw = [], [], []
        for k in range(NCTX):
            idx = ctxidx_v[k, pl.ds(c * CH, CH)]
            offc.append((idx & 1) * D)
            valc.append(jnp.where(idx == V, 0.0, 1.0))
        for r in range(NWN):
            idx = wnidx_v[r, pl.ds(c * CH, CH)]
            offw.append((idx & 1) * D)

        def dbody(d, accs):
            col = (d + iota) & (D - 1)
            csum = plsc.load_gather(ctxbuf, [rows_c[0], col + offc[0]]) * valc[0]
            for k in range(1, NCTX):
                v = plsc.load_gather(ctxbuf, [rows_c[k], col + offc[k]])
                csum = csum + v * valc[k]
            out = []
            for r in range(NWN):
                v = plsc.load_gather(wnbuf, [rows_w[r], col + offw[r]])
                out.append(accs[r] + csum * v)
            return tuple(out)

        zero = jnp.zeros((16,), jnp.float32)
        accs = lax.fori_loop(0, D, dbody, (zero,) * NWN)

        inv_len = 1.0 / lens_v[pl.ds(c * CH, CH)]
        pos = _sig_clamped(accs[0] * inv_len)
        loss = loss_acc + 0.5 * (1.0 - pos) * (1.0 - pos)
        for r in range(NEG):
            neg = _sig_clamped(accs[1 + r] * inv_len)
            neg = neg * mask_v[r, pl.ds(c * CH, CH)]
            loss = loss + 0.5 * neg * neg
        return loss

    loss = lax.fori_loop(0, NCH, chunk, jnp.zeros((16,), jnp.float32))
    lossbuf[...] = loss
    pltpu.sync_copy(lossbuf, out_hbm.at[wid])


@jax.jit
def kernel(emb0_weight, emb1_weight, data):
    d32 = data.astype(jnp.int32)
    ctx = d32[:, :NCTX]
    lens = d32[:, NCTX].astype(jnp.float32)
    wn = d32[:, NCTX + 1 : NCTX + 1 + NWN]
    mask = d32[:, NCTX + 1 + NWN :].astype(jnp.float32)

    e0t = emb0_weight.T
    e1t = emb1_weight.T

    mesh = plsc.VectorSubcoreMesh(core_axis_name="c", subcore_axis_name="s")

    run_t = pl.kernel(
        _transpose_body,
        out_type=(
            jax.ShapeDtypeStruct((PR, 2 * D), jnp.float32),
            jax.ShapeDtypeStruct((PR, 2 * D), jnp.float32),
        ),
        mesh=mesh,
        compiler_params=_PARAMS,
        scratch_types=[
            pltpu.VMEM((D, 2 * TCH), jnp.float32),
            pltpu.VMEM((D, 2 * TCH), jnp.float32),
            pltpu.VMEM((TCH, 2 * D), jnp.float32),
            pltpu.VMEM((TCH, 2 * D), jnp.float32),
            pltpu.VMEM((D, 2 * TAIL_P + 1), jnp.float32),
            pltpu.VMEM((D, 2 * TAIL_P), jnp.float32),
            pltpu.SemaphoreType.DMA,
            pltpu.SemaphoreType.DMA,
            pltpu.SemaphoreType.DMA,
            pltpu.SemaphoreType.DMA,
        ],
    )
    emb0_r, emb1_r = run_t(e0t, e1t)

    ctx_t = ctx.T.reshape(NCTX, NW, BW).transpose(1, 0, 2)
    wn_t = wn.T.reshape(NWN, NW, BW).transpose(1, 0, 2)
    lens_r = lens.reshape(NW, BW)
    mask_r = mask.T.reshape(NEG, NW, BW).transpose(1, 0, 2)

    run = pl.kernel(
        _cbow_body,
        out_type=jax.ShapeDtypeStruct((NW, 16), jnp.float32),
        mesh=mesh,
        compiler_params=_PARAMS,
        scratch_types=[
            pltpu.VMEM((NCTX, BW), jnp.int32),
            pltpu.VMEM((NWN, BW), jnp.int32),
            pltpu.VMEM((NCH * 2, CTX_PER_CH // 2), jnp.int32),
            pltpu.VMEM((NCH, WN_PER_CH), jnp.int32),
            pltpu.VMEM((BW,), jnp.float32),
            pltpu.VMEM((NEG, BW), jnp.float32),
            pltpu.VMEM((CTX_PER_CH, 2 * D), jnp.float32),
            pltpu.VMEM((WN_PER_CH, 2 * D), jnp.float32),
            pltpu.VMEM((16,), jnp.float32),
            pltpu.SemaphoreType.DMA,
        ],
    )
    partials = run(emb0_r, emb1_r, ctx_t, wn_t, lens_r, mask_r)
    return jnp.sum(partials)


# speedup vs baseline: 2.2879x; 1.0044x over previous
import functools

import jax
import jax.numpy as jnp
from jax import lax
from jax.experimental import pallas as pl
from jax.experimental.pallas import tpu as pltpu
from jax.experimental.pallas import tpu_sc as plsc

V = 1000000
D = 64
W = 5
NEG = 5
B = 16384
NCTX = 2 * W
NWN = 1 + NEG

_INFO = plsc.get_sparse_core_info()
NC = _INFO.num_cores
NS = _INFO.num_subcores
NW = NC * NS
BW = B // NW
CH = 16
NCH = BW // CH
CTX_PER_CH = CH * NCTX
WN_PER_CH = CH * NWN

PR = V // 2
TCH = 64
NG = PR // TCH
NG_PW = NG // NW
NG_EXTRA = NG - NG_PW * NW
TAIL_P = PR - NG * TCH
TAIL_C0 = NG * 2 * TCH

_PARAMS = pltpu.CompilerParams(
    needs_layout_passes=False, use_tc_tiling_on_sc=True
)


def _sig_clamped(x):
    s = 1.0 / (1.0 + jnp.exp(-x))
    s = jnp.where(x > 6.0, 1.0, s)
    return jnp.where(x > -6.0, s, 0.0)


def _transpose_body(e0t_hbm, e1t_hbm, out0_hbm, out1_hbm,
                    strip0, strip1, obuf0, obuf1, tstrip0, tstrip1,
                    s_in0, s_in1, s_out0, s_out1):
    wid = lax.axis_index("s") * NC + lax.axis_index("c")

    iota = lax.iota(jnp.int32, 16)
    sq = [16 * q + iota for q in range(4)]
    hsq = [[64 * h + 16 * q + iota for q in range(4)] for h in range(2)]
    ia7 = iota & 7
    h_a = iota >> 3
    h_b = 1 - h_a
    cola = [h_a * 64 + 16 * q + iota for q in range(4)]
    colb = [h_b * 64 + 16 * q + iota for q in range(4)]
    nfull = jnp.where(wid < NG_EXTRA, NG_PW + 1, NG_PW)

    def one_table(src, dst):
        def fire_in(j, strip, sem):
            gid = wid + NW * j
            pltpu.async_copy(src.at[:, pl.ds(gid * 2 * TCH, 2 * TCH)],
                             strip, sem)

        def wait_in(strip, sem):
            pltpu.make_async_copy(src.at[:, pl.ds(0, 2 * TCH)], strip,
                                  sem).wait()

        def fire_out(j, obuf, sem):
            gid = wid + NW * j
            pltpu.async_copy(obuf, dst.at[pl.ds(gid * TCH, TCH)], sem)

        def wait_out(obuf, sem):
            pltpu.make_async_copy(obuf, dst.at[pl.ds(0, TCH)], sem).wait()

        def compute(strip, obuf):
            def prow(u, carry):
                vals, tees = [], []
                for pp in range(4):
                    p0 = u * 4 + pp
                    t = (p0 + ia7) & (TCH - 1)
                    c2 = t << 1
                    for hv, colq in ((h_a, cola), (h_b, colb)):
                        cv = c2 + hv
                        for q in range(4):
                            vals.append(
                                plsc.load_gather(strip, [sq[q], cv]))
                            tees.append((t, colq[q]))
                for v, (t, cols) in zip(vals, tees):
                    plsc.store_scatter(obuf, [t, cols], v)
                return carry

            lax.fori_loop(0, TCH // 4, prow, 0)

        def chunk(j, carry):
            even = j % 2 == 0

            @pl.when(j + 1 < nfull)
            def _():
                @pl.when(even)
                def _():
                    fire_in(j + 1, strip1, s_in1)

                @pl.when(jnp.logical_not(even))
                def _():
                    fire_in(j + 1, strip0, s_in0)

            @pl.when(even)
            def _():
                wait_in(strip0, s_in0)

                @pl.when(j >= 2)
                def _():
                    wait_out(obuf0, s_out0)

                compute(strip0, obuf0)
                fire_out(j, obuf0, s_out0)

            @pl.when(jnp.logical_not(even))
            def _():
                wait_in(strip1, s_in1)

                @pl.when(j >= 2)
                def _():
                    wait_out(obuf1, s_out1)

                compute(strip1, obuf1)
                fire_out(j, obuf1, s_out1)

            return carry

        fire_in(0, strip0, s_in0)
        lax.fori_loop(0, nfull, chunk, 0)
        wait_out(obuf1, s_out1)
        wait_out(obuf0, s_out0)

    one_table(e0t_hbm, out0_hbm)
    one_table(e1t_hbm, out1_hbm)

    def tail(src_strip, obuf, dst):
        def prow(p, carry):
            for h in range(2):
                cv = 2 * p + h + iota * 0
                for q in range(4):
                    v = plsc.load_gather(src_strip, [sq[q], cv])
                    obuf[p, pl.ds(64 * h + 16 * q, 16)] = v
            return carry

        lax.fori_loop(0, TAIL_P, prow, 0)
        pltpu.sync_copy(obuf.at[pl.ds(0, TAIL_P)],
                        dst.at[pl.ds(NG * TCH, TAIL_P)])

    @pl.when(wid == 0)
    def _():
        pltpu.sync_copy(e0t_hbm.at[:, pl.ds(TAIL_C0, 2 * TAIL_P + 1)], tstrip0)
        tail(tstrip0, obuf0, out0_hbm)

    @pl.when(wid == 1)
    def _():
        pltpu.sync_copy(e1t_hbm.at[:, pl.ds(TAIL_C0, 2 * TAIL_P)], tstrip1)
        tail(tstrip1, obuf1, out1_hbm)


def _cbow_body(emb0_hbm, emb1_hbm, ctx_t_hbm, wn_t_hbm, lens_hbm, mask_hbm,
               out_hbm,
               ctxidx_v, wnidx_v, physctx, physwn, lens_v, mask_v,
               ctxbuf, wnbuf, lossbuf, sem):
    wid = lax.axis_index("s") * NC + lax.axis_index("c")

    pltpu.sync_copy(ctx_t_hbm.at[wid], ctxidx_v)
    pltpu.sync_copy(wn_t_hbm.at[wid], wnidx_v)
    pltpu.sync_copy(lens_hbm.at[wid], lens_v)
    pltpu.sync_copy(mask_hbm.at[wid], mask_v)

    iota = lax.iota(jnp.int32, 16)

    def prep(c, carry):
        for k in range(NCTX):
            idx = ctxidx_v[k, pl.ds(c * CH, CH)]
            phys = jnp.where(idx == V, 0, idx >> 1)
            physctx[2 * c + k // 5, pl.ds((k % 5) * CH, CH)] = phys
        for r in range(NWN):
            idx = wnidx_v[r, pl.ds(c * CH, CH)]
            physwn[c, pl.ds(r * CH, CH)] = idx >> 1
        return carry

    lax.fori_loop(0, NCH, prep, 0)

    rows_c = [iota + CH * k for k in range(NCTX)]
    rows_w = [iota + CH * r for r in range(NWN)]

    def chunk(c, loss_acc):
        pltpu.async_copy(emb0_hbm.at[physctx.at[2 * c]],
                         ctxbuf.at[pl.ds(0, 80)], sem)
        pltpu.async_copy(emb0_hbm.at[physctx.at[2 * c + 1]],
                         ctxbuf.at[pl.ds(80, 80)], sem)
        pltpu.async_copy(emb1_hbm.at[physwn.at[c]], wnbuf, sem).wait()
        pltpu.make_async_copy(emb0_hbm.at[pl.ds(0, CTX_PER_CH)], ctxbuf,
                              sem).wait()

        offc, valc, offw = [], [], []
        for k in range(NCTX):
            idx = ctxidx_v[k, pl.ds(c * CH, CH)]
            offc.append((idx & 1) * D)
            valc.append(jnp.where(idx == V, 0.0, 1.0))
        for r in range(NWN):
            idx = wnidx_v[r, pl.ds(c * CH, CH)]
            offw.append((idx & 1) * D)

        def dbody(d, accs):
            col = (d + iota) & (D - 1)
            csum = plsc.load_gather(ctxbuf, [rows_c[0], col + offc[0]]) * valc[0]
            for k in range(1, NCTX):
                v = plsc.load_gather(ctxbuf, [rows_c[k], col + offc[k]])
                csum = csum + v * valc[k]
            out = []
            for r in range(NWN):
                v = plsc.load_gather(wnbuf, [rows_w[r], col + offw[r]])
                out.append(accs[r] + csum * v)
            return tuple(out)

        zero = jnp.zeros((16,), jnp.float32)
        accs = lax.fori_loop(0, D, dbody, (zero,) * NWN)

        inv_len = 1.0 / lens_v[pl.ds(c * CH, CH)]
        pos = _sig_clamped(accs[0] * inv_len)
        loss = loss_acc + 0.5 * (1.0 - pos) * (1.0 - pos)
        for r in range(NEG):
            neg = _sig_clamped(accs[1 + r] * inv_len)
            neg = neg * mask_v[r, pl.ds(c * CH, CH)]
            loss = loss + 0.5 * neg * neg
        return loss

    loss = lax.fori_loop(0, NCH, chunk, jnp.zeros((16,), jnp.float32))
    lossbuf[...] = loss
    pltpu.sync_copy(lossbuf, out_hbm.at[wid])


@jax.jit
def kernel(emb0_weight, emb1_weight, data):
    d32 = data.astype(jnp.int32)
    ctx = d32[:, :NCTX]
    lens = d32[:, NCTX].astype(jnp.float32)
    wn = d32[:, NCTX + 1 : NCTX + 1 + NWN]
    mask = d32[:, NCTX + 1 + NWN :].astype(jnp.float32)

    e0t = emb0_weight.T
    e1t = emb1_weight.T

    mesh = plsc.VectorSubcoreMesh(core_axis_name="c", subcore_axis_name="s")

    run_t = pl.kernel(
        _transpose_body,
        out_type=(
            jax.ShapeDtypeStruct((PR, 2 * D), jnp.float32),
            jax.ShapeDtypeStruct((PR, 2 * D), jnp.float32),
        ),
        mesh=mesh,
        compiler_params=_PARAMS,
        scratch_types=[
            pltpu.VMEM((D, 2 * TCH), jnp.float32),
            pltpu.VMEM((D, 2 * TCH), jnp.float32),
            pltpu.VMEM((TCH, 2 * D), jnp.float32),
            pltpu.VMEM((TCH, 2 * D), jnp.float32),
            pltpu.VMEM((D, 2 * TAIL_P + 1), jnp.float32),
            pltpu.VMEM((D, 2 * TAIL_P), jnp.float32),
            pltpu.SemaphoreType.DMA,
            pltpu.SemaphoreType.DMA,
            pltpu.SemaphoreType.DMA,
            pltpu.SemaphoreType.DMA,
        ],
    )
    emb0_r, emb1_r = run_t(e0t, e1t)

    ctx_t = ctx.T.reshape(NCTX, NW, BW).transpose(1, 0, 2)
    wn_t = wn.T.reshape(NWN, NW, BW).transpose(1, 0, 2)
    lens_r = lens.reshape(NW, BW)
    mask_r = mask.T.reshape(NEG, NW, BW).transpose(1, 0, 2)

    run = pl.kernel(
        _cbow_body,
        out_type=jax.ShapeDtypeStruct((NW, 16), jnp.float32),
        mesh=mesh,
        compiler_params=_PARAMS,
        scratch_types=[
            pltpu.VMEM((NCTX, BW), jnp.int32),
            pltpu.VMEM((NWN, BW), jnp.int32),
            pltpu.VMEM((NCH * 2, CTX_PER_CH // 2), jnp.int32),
            pltpu.VMEM((NCH, WN_PER_CH), jnp.int32),
            pltpu.VMEM((BW,), jnp.float32),
            pltpu.VMEM((NEG, BW), jnp.float32),
            pltpu.VMEM((CTX_PER_CH, 2 * D), jnp.float32),
            pltpu.VMEM((WN_PER_CH, 2 * D), jnp.float32),
            pltpu.VMEM((16,), jnp.float32),
            pltpu.SemaphoreType.DMA,
        ],
    )
    partials = run(emb0_r, emb1_r, ctx_t, wn_t, lens_r, mask_r)
    return jnp.sum(partials)


# speedup vs baseline: 2.4281x; 1.0613x over previous
import functools

import jax
import jax.numpy as jnp
from jax import lax
from jax.experimental import pallas as pl
from jax.experimental.pallas import tpu as pltpu
from jax.experimental.pallas import tpu_sc as plsc

V = 1000000
D = 64
W = 5
NEG = 5
B = 16384
NCTX = 2 * W
NWN = 1 + NEG

_INFO = plsc.get_sparse_core_info()
NC = _INFO.num_cores
NS = _INFO.num_subcores
NW = NC * NS
BW = B // NW
CH = 16
NCH = BW // CH
CTX_PER_CH = CH * NCTX
WN_PER_CH = CH * NWN

PR = V // 2
TCH = 64
NG = PR // TCH
NG_PW = NG // NW
NG_EXTRA = NG - NG_PW * NW
TAIL_P = PR - NG * TCH
TAIL_C0 = NG * 2 * TCH

_PARAMS = pltpu.CompilerParams(
    needs_layout_passes=False, use_tc_tiling_on_sc=True
)


def _sig_clamped(x):
    s = 1.0 / (1.0 + jnp.exp(-x))
    s = jnp.where(x > 6.0, 1.0, s)
    return jnp.where(x > -6.0, s, 0.0)


def _transpose_body(e0t_hbm, e1t_hbm, out0_hbm, out1_hbm,
                    strip0, strip1, obuf0, obuf1, tstrip0, tstrip1,
                    s_in0, s_in1, s_out0, s_out1):
    wid = lax.axis_index("s") * NC + lax.axis_index("c")

    iota = lax.iota(jnp.int32, 16)
    sq = [16 * q + iota for q in range(4)]
    hsq = [[64 * h + 16 * q + iota for q in range(4)] for h in range(2)]
    ia7 = iota & 7
    h_a = iota >> 3
    h_b = 1 - h_a
    cola = [h_a * 64 + 16 * q + iota for q in range(4)]
    colb = [h_b * 64 + 16 * q + iota for q in range(4)]
    nfull = jnp.where(wid < NG_EXTRA, NG_PW + 1, NG_PW)

    def one_table(src, dst):
        def fire_in(j, strip, sem):
            gid = wid + NW * j
            pltpu.async_copy(src.at[:, pl.ds(gid * 2 * TCH, 2 * TCH)],
                             strip, sem)

        def wait_in(strip, sem):
            pltpu.make_async_copy(src.at[:, pl.ds(0, 2 * TCH)], strip,
                                  sem).wait()

        def fire_out(j, obuf, sem):
            gid = wid + NW * j
            pltpu.async_copy(obuf, dst.at[pl.ds(gid * TCH, TCH)], sem)

        def wait_out(obuf, sem):
            pltpu.make_async_copy(obuf, dst.at[pl.ds(0, TCH)], sem).wait()

        def compute(strip, obuf):
            def prow(u, carry):
                vals, tees = [], []
                for pp in range(4):
                    p0 = u * 4 + pp
                    t = (p0 + ia7) & (TCH - 1)
                    c2 = t << 1
                    for hv, colq in ((h_a, cola), (h_b, colb)):
                        cv = c2 + hv
                        for q in range(4):
                            vals.append(
                                plsc.load_gather(strip, [sq[q], cv]))
                            tees.append((t, colq[q]))
                for v, (t, cols) in zip(vals, tees):
                    plsc.store_scatter(obuf, [t, cols], v)
                return carry

            lax.fori_loop(0, TCH // 4, prow, 0)

        def chunk(j, carry):
            even = j % 2 == 0

            @pl.when(j + 1 < nfull)
            def _():
                @pl.when(even)
                def _():
                    fire_in(j + 1, strip1, s_in1)

                @pl.when(jnp.logical_not(even))
                def _():
                    fire_in(j + 1, strip0, s_in0)

            @pl.when(even)
            def _():
                wait_in(strip0, s_in0)

                @pl.when(j >= 2)
                def _():
                    wait_out(obuf0, s_out0)

                compute(strip0, obuf0)
                fire_out(j, obuf0, s_out0)

            @pl.when(jnp.logical_not(even))
            def _():
                wait_in(strip1, s_in1)

                @pl.when(j >= 2)
                def _():
                    wait_out(obuf1, s_out1)

                compute(strip1, obuf1)
                fire_out(j, obuf1, s_out1)

            return carry

        fire_in(0, strip0, s_in0)
        lax.fori_loop(0, nfull, chunk, 0)
        wait_out(obuf1, s_out1)
        wait_out(obuf0, s_out0)

    one_table(e0t_hbm, out0_hbm)
    one_table(e1t_hbm, out1_hbm)

    def tail(src_strip, obuf, dst):
        def prow(p, carry):
            for h in range(2):
                cv = 2 * p + h + iota * 0
                for q in range(4):
                    v = plsc.load_gather(src_strip, [sq[q], cv])
                    obuf[p, pl.ds(64 * h + 16 * q, 16)] = v
            return carry

        lax.fori_loop(0, TAIL_P, prow, 0)
        pltpu.sync_copy(obuf.at[pl.ds(0, TAIL_P)],
                        dst.at[pl.ds(NG * TCH, TAIL_P)])

    @pl.when(wid == 0)
    def _():
        pltpu.sync_copy(e0t_hbm.at[:, pl.ds(TAIL_C0, 2 * TAIL_P + 1)], tstrip0)
        tail(tstrip0, obuf0, out0_hbm)

    @pl.when(wid == 1)
    def _():
        pltpu.sync_copy(e1t_hbm.at[:, pl.ds(TAIL_C0, 2 * TAIL_P)], tstrip1)
        tail(tstrip1, obuf1, out1_hbm)


def _cbow_body(emb0_hbm, emb1_hbm, ctx_t_hbm, wn_t_hbm, lens_hbm, mask_hbm,
               out_hbm,
               ctxidx_v, wnidx_v, physctx, physwn, lens_v, mask_v,
               ctxbuf0, ctxbuf1, ctxbuf2, wnbuf0, wnbuf1, wnbuf2, lossbuf,
               sem0, sem1, sem2):
    wid = lax.axis_index("s") * NC + lax.axis_index("c")

    pltpu.sync_copy(ctx_t_hbm.at[wid], ctxidx_v)
    pltpu.sync_copy(wn_t_hbm.at[wid], wnidx_v)
    pltpu.sync_copy(lens_hbm.at[wid], lens_v)
    pltpu.sync_copy(mask_hbm.at[wid], mask_v)

    iota = lax.iota(jnp.int32, 16)

    def prep(c, carry):
        for k in range(NCTX):
            idx = ctxidx_v[k, pl.ds(c * CH, CH)]
            phys = jnp.where(idx == V, 0, idx >> 1)
            physctx[2 * c + k // 5, pl.ds((k % 5) * CH, CH)] = phys
        for r in range(NWN):
            idx = wnidx_v[r, pl.ds(c * CH, CH)]
            physwn[c, pl.ds(r * CH, CH)] = idx >> 1
        return carry

    lax.fori_loop(0, NCH, prep, 0)

    rows_c = [iota + CH * k for k in range(NCTX)]
    rows_w = [iota + CH * r for r in range(NWN)]

    bufs = ((ctxbuf0, wnbuf0, sem0), (ctxbuf1, wnbuf1, sem1),
            (ctxbuf2, wnbuf2, sem2))

    def fire(c, ctxbuf, wnbuf, sem):
        pltpu.async_copy(emb0_hbm.at[physctx.at[2 * c]],
                         ctxbuf.at[pl.ds(0, 80)], sem)
        pltpu.async_copy(emb0_hbm.at[physctx.at[2 * c + 1]],
                         ctxbuf.at[pl.ds(80, 80)], sem)
        pltpu.async_copy(emb1_hbm.at[physwn.at[c]], wnbuf, sem)

    def wait_bufs(ctxbuf, wnbuf, sem):
        pltpu.make_async_copy(emb1_hbm.at[pl.ds(0, WN_PER_CH)], wnbuf,
                              sem).wait()
        pltpu.make_async_copy(emb0_hbm.at[pl.ds(0, CTX_PER_CH)], ctxbuf,
                              sem).wait()

    def compute(c, ctxbuf, wnbuf):
        offc, valc, offw = [], [], []
        for k in range(NCTX):
            idx = ctxidx_v[k, pl.ds(c * CH, CH)]
            offc.append((idx & 1) * D)
            valc.append(jnp.where(idx == V, 0.0, 1.0))
        for r in range(NWN):
            idx = wnidx_v[r, pl.ds(c * CH, CH)]
            offw.append((idx & 1) * D)

        def dbody(d, accs):
            col = (d + iota) & (D - 1)
            csum = plsc.load_gather(ctxbuf, [rows_c[0], col + offc[0]]) * valc[0]
            for k in range(1, NCTX):
                v = plsc.load_gather(ctxbuf, [rows_c[k], col + offc[k]])
                csum = csum + v * valc[k]
            out = []
            for r in range(NWN):
                v = plsc.load_gather(wnbuf, [rows_w[r], col + offw[r]])
                out.append(accs[r] + csum * v)
            return tuple(out)

        zero = jnp.zeros((16,), jnp.float32)
        accs = lax.fori_loop(0, D, dbody, (zero,) * NWN)

        inv_len = 1.0 / lens_v[pl.ds(c * CH, CH)]
        pos = _sig_clamped(accs[0] * inv_len)
        loss = 0.5 * (1.0 - pos) * (1.0 - pos)
        for r in range(NEG):
            neg = _sig_clamped(accs[1 + r] * inv_len)
            neg = neg * mask_v[r, pl.ds(c * CH, CH)]
            loss = loss + 0.5 * neg * neg
        lossbuf[...] = lossbuf[...] + loss

    lossbuf[...] = jnp.zeros((16,), jnp.float32)
    fire(0, *bufs[0])
    fire(1, *bufs[1])

    def chunk(c, carry):
        for m in range(3):
            @pl.when((c + 2 < NCH) & ((c + 2) % 3 == m))
            def _(m=m):
                fire(c + 2, *bufs[m])

        for m in range(3):
            @pl.when(c % 3 == m)
            def _(m=m):
                ctxbuf, wnbuf, sem = bufs[m]
                wait_bufs(ctxbuf, wnbuf, sem)
                compute(c, ctxbuf, wnbuf)

        return carry

    lax.fori_loop(0, NCH, chunk, 0)
    pltpu.sync_copy(lossbuf, out_hbm.at[wid])


@jax.jit
def kernel(emb0_weight, emb1_weight, data):
    d32 = data.astype(jnp.int32)
    ctx = d32[:, :NCTX]
    lens = d32[:, NCTX].astype(jnp.float32)
    wn = d32[:, NCTX + 1 : NCTX + 1 + NWN]
    mask = d32[:, NCTX + 1 + NWN :].astype(jnp.float32)

    e0t = emb0_weight.T
    e1t = emb1_weight.T

    mesh = plsc.VectorSubcoreMesh(core_axis_name="c", subcore_axis_name="s")

    run_t = pl.kernel(
        _transpose_body,
        out_type=(
            jax.ShapeDtypeStruct((PR, 2 * D), jnp.float32),
            jax.ShapeDtypeStruct((PR, 2 * D), jnp.float32),
        ),
        mesh=mesh,
        compiler_params=_PARAMS,
        scratch_types=[
            pltpu.VMEM((D, 2 * TCH), jnp.float32),
            pltpu.VMEM((D, 2 * TCH), jnp.float32),
            pltpu.VMEM((TCH, 2 * D), jnp.float32),
            pltpu.VMEM((TCH, 2 * D), jnp.float32),
            pltpu.VMEM((D, 2 * TAIL_P + 1), jnp.float32),
            pltpu.VMEM((D, 2 * TAIL_P), jnp.float32),
            pltpu.SemaphoreType.DMA,
            pltpu.SemaphoreType.DMA,
            pltpu.SemaphoreType.DMA,
            pltpu.SemaphoreType.DMA,
        ],
    )
    emb0_r, emb1_r = run_t(e0t, e1t)

    ctx_t = ctx.T.reshape(NCTX, NW, BW).transpose(1, 0, 2)
    wn_t = wn.T.reshape(NWN, NW, BW).transpose(1, 0, 2)
    lens_r = lens.reshape(NW, BW)
    mask_r = mask.T.reshape(NEG, NW, BW).transpose(1, 0, 2)

    run = pl.kernel(
        _cbow_body,
        out_type=jax.ShapeDtypeStruct((NW, 16), jnp.float32),
        mesh=mesh,
        compiler_params=_PARAMS,
        scratch_types=[
            pltpu.VMEM((NCTX, BW), jnp.int32),
            pltpu.VMEM((NWN, BW), jnp.int32),
            pltpu.VMEM((NCH * 2, CTX_PER_CH // 2), jnp.int32),
            pltpu.VMEM((NCH, WN_PER_CH), jnp.int32),
            pltpu.VMEM((BW,), jnp.float32),
            pltpu.VMEM((NEG, BW), jnp.float32),
            pltpu.VMEM((CTX_PER_CH, 2 * D), jnp.float32),
            pltpu.VMEM((CTX_PER_CH, 2 * D), jnp.float32),
            pltpu.VMEM((CTX_PER_CH, 2 * D), jnp.float32),
            pltpu.VMEM((WN_PER_CH, 2 * D), jnp.float32),
            pltpu.VMEM((WN_PER_CH, 2 * D), jnp.float32),
            pltpu.VMEM((WN_PER_CH, 2 * D), jnp.float32),
            pltpu.VMEM((16,), jnp.float32),
            pltpu.SemaphoreType.DMA,
            pltpu.SemaphoreType.DMA,
            pltpu.SemaphoreType.DMA,
        ],
    )
    partials = run(emb0_r, emb1_r, ctx_t, wn_t, lens_r, mask_r)
    return jnp.sum(partials)


# speedup vs baseline: 2.7212x; 1.1207x over previous
import functools

import jax
import jax.numpy as jnp
from jax import lax
from jax.experimental import pallas as pl
from jax.experimental.pallas import tpu as pltpu
from jax.experimental.pallas import tpu_sc as plsc

V = 1000000
D = 64
W = 5
NEG = 5
B = 16384
NCTX = 2 * W
NWN = 1 + NEG

_INFO = plsc.get_sparse_core_info()
NC = _INFO.num_cores
NS = _INFO.num_subcores
NW = NC * NS
BW = B // NW
CH = 16
NCH = BW // CH
CTX_PER_CH = CH * NCTX
WN_PER_CH = CH * NWN

SCOLS = 128
PR = V // 4
OCH = SCOLS // 4
NG = V // SCOLS
NG_PW = NG // NW
NG_EXTRA = NG - NG_PW * NW
TAIL_L = V - NG * SCOLS
TAIL_C0 = NG * SCOLS
MASK_HI = -65536

_PARAMS = pltpu.CompilerParams(
    needs_layout_passes=False, use_tc_tiling_on_sc=True
)


def _sig_clamped(x):
    s = 1.0 / (1.0 + jnp.exp(-x))
    s = jnp.where(x > 6.0, 1.0, s)
    return jnp.where(x > -6.0, s, 0.0)


def _transpose_body(e0t_hbm, e1t_hbm, out0_hbm, out1_hbm,
                    strip0, strip1, obuf0, obuf1, tstrip0, tstrip1,
                    s_in0, s_in1, s_out0, s_out1):
    wid = lax.axis_index("s") * NC + lax.axis_index("c")

    iota = lax.iota(jnp.int32, 16)
    rev = [2 * iota + 32 * q2 for q2 in range(2)]
    cq2 = [16 * q2 + iota for q2 in range(2)]
    ia7 = iota & 7
    h_a = iota >> 3
    h_b = 1 - h_a
    nfull = jnp.where(wid < NG_EXTRA, NG_PW + 1, NG_PW)

    def pack_words(strip, lv, q2):
        e = plsc.load_gather(strip, [rev[q2], lv])
        o = plsc.load_gather(strip, [rev[q2] + 1, lv])
        ie = plsc.bitcast(e, jnp.int32)
        io = plsc.bitcast(o, jnp.int32)
        return ((ie >> 16) & 0xFFFF) | (io & MASK_HI)

    def one_table(src, dst):
        def fire_in(j, strip, sem):
            gid = wid + NW * j
            pltpu.async_copy(src.at[:, pl.ds(gid * SCOLS, SCOLS)],
                             strip, sem)

        def wait_in(strip, sem):
            pltpu.make_async_copy(src.at[:, pl.ds(0, SCOLS)], strip,
                                  sem).wait()

        def fire_out(j, obuf, sem):
            gid = wid + NW * j
            pltpu.async_copy(obuf, dst.at[pl.ds(gid * OCH, OCH)], sem)

        def wait_out(obuf, sem):
            pltpu.make_async_copy(obuf, dst.at[pl.ds(0, OCH)], sem).wait()

        def compute(strip, obuf):
            def prow(u, carry):
                vals, tees = [], []
                for pp in range(4):
                    p0 = u * 4 + pp
                    t = (p0 + ia7) & (SCOLS // 2 - 1)
                    c2 = t << 1
                    for hv in (h_a, h_b):
                        lv = c2 + hv
                        rowv = lv >> 2
                        colbase = (lv & 3) << 5
                        for q2 in range(2):
                            vals.append(pack_words(strip, lv, q2))
                            tees.append((rowv, colbase + cq2[q2]))
                for w, (rowv, cols) in zip(vals, tees):
                    plsc.store_scatter(obuf, [rowv, cols], w)
                return carry

            lax.fori_loop(0, SCOLS // 8, prow, 0)

        def chunk(j, carry):
            even = j % 2 == 0

            @pl.when(j + 1 < nfull)
            def _():
                @pl.when(even)
                def _():
                    fire_in(j + 1, strip1, s_in1)

                @pl.when(jnp.logical_not(even))
                def _():
                    fire_in(j + 1, strip0, s_in0)

            @pl.when(even)
            def _():
                wait_in(strip0, s_in0)

                @pl.when(j >= 2)
                def _():
                    wait_out(obuf0, s_out0)

                compute(strip0, obuf0)
                fire_out(j, obuf0, s_out0)

            @pl.when(jnp.logical_not(even))
            def _():
                wait_in(strip1, s_in1)

                @pl.when(j >= 2)
                def _():
                    wait_out(obuf1, s_out1)

                compute(strip1, obuf1)
                fire_out(j, obuf1, s_out1)

            return carry

        fire_in(0, strip0, s_in0)
        lax.fori_loop(0, nfull, chunk, 0)
        wait_out(obuf1, s_out1)
        wait_out(obuf0, s_out0)

    one_table(e0t_hbm, out0_hbm)
    one_table(e1t_hbm, out1_hbm)

    def tail(src_strip, obuf, dst):
        def prow(l, carry):
            lv = l + iota * 0
            for q2 in range(2):
                w = pack_words(src_strip, lv, q2)
                obuf[l >> 2, pl.ds((l & 3) * 32 + 16 * q2, 16)] = w
            return carry

        lax.fori_loop(0, TAIL_L, prow, 0)
        pltpu.sync_copy(obuf.at[pl.ds(0, TAIL_L // 4)],
                        dst.at[pl.ds(NG * OCH, TAIL_L // 4)])

    @pl.when(wid == 0)
    def _():
        pltpu.sync_copy(e0t_hbm.at[:, pl.ds(TAIL_C0, TAIL_L + 1)], tstrip0)
        tail(tstrip0, obuf0, out0_hbm)

    @pl.when(wid == 1)
    def _():
        pltpu.sync_copy(e1t_hbm.at[:, pl.ds(TAIL_C0, TAIL_L)], tstrip1)
        tail(tstrip1, obuf1, out1_hbm)


def _cbow_body(emb0_hbm, emb1_hbm, ctx_t_hbm, wn_t_hbm, lens_hbm, mask_hbm,
               out_hbm,
               ctxidx_v, wnidx_v, physctx, physwn, lens_v, mask_v,
               ctxbuf0, ctxbuf1, ctxbuf2, wnbuf0, wnbuf1, wnbuf2, lossbuf,
               sem0, sem1, sem2):
    wid = lax.axis_index("s") * NC + lax.axis_index("c")

    pltpu.sync_copy(ctx_t_hbm.at[wid], ctxidx_v)
    pltpu.sync_copy(wn_t_hbm.at[wid], wnidx_v)
    pltpu.sync_copy(lens_hbm.at[wid], lens_v)
    pltpu.sync_copy(mask_hbm.at[wid], mask_v)

    iota = lax.iota(jnp.int32, 16)

    def prep(c, carry):
        for k in range(NCTX):
            idx = ctxidx_v[k, pl.ds(c * CH, CH)]
            phys = jnp.where(idx == V, 0, idx >> 2)
            physctx[2 * c + k // 5, pl.ds((k % 5) * CH, CH)] = phys
        for r in range(NWN):
            idx = wnidx_v[r, pl.ds(c * CH, CH)]
            physwn[c, pl.ds(r * CH, CH)] = idx >> 2
        return carry

    lax.fori_loop(0, NCH, prep, 0)

    rows_c = [iota + CH * k for k in range(NCTX)]
    rows_w = [iota + CH * r for r in range(NWN)]

    bufs = ((ctxbuf0, wnbuf0, sem0), (ctxbuf1, wnbuf1, sem1),
            (ctxbuf2, wnbuf2, sem2))

    def fire(c, ctxbuf, wnbuf, sem):
        pltpu.async_copy(emb0_hbm.at[physctx.at[2 * c]],
                         ctxbuf.at[pl.ds(0, 80)], sem)
        pltpu.async_copy(emb0_hbm.at[physctx.at[2 * c + 1]],
                         ctxbuf.at[pl.ds(80, 80)], sem)
        pltpu.async_copy(emb1_hbm.at[physwn.at[c]], wnbuf, sem)

    def wait_bufs(ctxbuf, wnbuf, sem):
        pltpu.make_async_copy(emb1_hbm.at[pl.ds(0, WN_PER_CH)], wnbuf,
                              sem).wait()
        pltpu.make_async_copy(emb0_hbm.at[pl.ds(0, CTX_PER_CH)], ctxbuf,
                              sem).wait()

    def unpack2(w):
        lo = plsc.bitcast(w << 16, jnp.float32)
        hi = plsc.bitcast(w & MASK_HI, jnp.float32)
        return lo, hi

    def compute(c, ctxbuf, wnbuf):
        offc, valc, offw = [], [], []
        for k in range(NCTX):
            idx = ctxidx_v[k, pl.ds(c * CH, CH)]
            offc.append((idx & 3) * 32)
            valc.append(jnp.where(idx == V, 0.0, 1.0))
        for r in range(NWN):
            idx = wnidx_v[r, pl.ds(c * CH, CH)]
            offw.append((idx & 3) * 32)

        def dbody(d, accs):
            col = (d + iota) & 31
            csl = jnp.zeros((16,), jnp.float32)
            csh = jnp.zeros((16,), jnp.float32)
            for k in range(NCTX):
                w = plsc.load_gather(ctxbuf, [rows_c[k], col + offc[k]])
                lo, hi = unpack2(w)
                csl = csl + lo * valc[k]
                csh = csh + hi * valc[k]
            out = []
            for r in range(NWN):
                w = plsc.load_gather(wnbuf, [rows_w[r], col + offw[r]])
                lo, hi = unpack2(w)
                out.append(accs[r] + csl * lo + csh * hi)
            return tuple(out)

        zero = jnp.zeros((16,), jnp.float32)
        accs = lax.fori_loop(0, D // 2, dbody, (zero,) * NWN)

        inv_len = 1.0 / lens_v[pl.ds(c * CH, CH)]
        pos = _sig_clamped(accs[0] * inv_len)
        loss = 0.5 * (1.0 - pos) * (1.0 - pos)
        for r in range(NEG):
            neg = _sig_clamped(accs[1 + r] * inv_len)
            neg = neg * mask_v[r, pl.ds(c * CH, CH)]
            loss = loss + 0.5 * neg * neg
        lossbuf[...] = lossbuf[...] + loss

    lossbuf[...] = jnp.zeros((16,), jnp.float32)
    fire(0, *bufs[0])
    fire(1, *bufs[1])

    def chunk(c, carry):
        for m in range(3):
            @pl.when((c + 2 < NCH) & ((c + 2) % 3 == m))
            def _(m=m):
                fire(c + 2, *bufs[m])

        for m in range(3):
            @pl.when(c % 3 == m)
            def _(m=m):
                ctxbuf, wnbuf, sem = bufs[m]
                wait_bufs(ctxbuf, wnbuf, sem)
                compute(c, ctxbuf, wnbuf)

        return carry

    lax.fori_loop(0, NCH, chunk, 0)
    pltpu.sync_copy(lossbuf, out_hbm.at[wid])


@jax.jit
def kernel(emb0_weight, emb1_weight, data):
    d32 = data.astype(jnp.int32)
    ctx = d32[:, :NCTX]
    lens = d32[:, NCTX].astype(jnp.float32)
    wn = d32[:, NCTX + 1 : NCTX + 1 + NWN]
    mask = d32[:, NCTX + 1 + NWN :].astype(jnp.float32)

    e0t = emb0_weight.T
    e1t = emb1_weight.T

    mesh = plsc.VectorSubcoreMesh(core_axis_name="c", subcore_axis_name="s")

    run_t = pl.kernel(
        _transpose_body,
        out_type=(
            jax.ShapeDtypeStruct((PR, 2 * D), jnp.int32),
            jax.ShapeDtypeStruct((PR, 2 * D), jnp.int32),
        ),
        mesh=mesh,
        compiler_params=_PARAMS,
        scratch_types=[
            pltpu.VMEM((D, SCOLS), jnp.float32),
            pltpu.VMEM((D, SCOLS), jnp.float32),
            pltpu.VMEM((OCH, 2 * D), jnp.int32),
            pltpu.VMEM((OCH, 2 * D), jnp.int32),
            pltpu.VMEM((D, TAIL_L + 1), jnp.float32),
            pltpu.VMEM((D, TAIL_L), jnp.float32),
            pltpu.SemaphoreType.DMA,
            pltpu.SemaphoreType.DMA,
            pltpu.SemaphoreType.DMA,
            pltpu.SemaphoreType.DMA,
        ],
    )
    emb0_r, emb1_r = run_t(e0t, e1t)

    ctx_t = ctx.T.reshape(NCTX, NW, BW).transpose(1, 0, 2)
    wn_t = wn.T.reshape(NWN, NW, BW).transpose(1, 0, 2)
    lens_r = lens.reshape(NW, BW)
    mask_r = mask.T.reshape(NEG, NW, BW).transpose(1, 0, 2)

    run = pl.kernel(
        _cbow_body,
        out_type=jax.ShapeDtypeStruct((NW, 16), jnp.float32),
        mesh=mesh,
        compiler_params=_PARAMS,
        scratch_types=[
            pltpu.VMEM((NCTX, BW), jnp.int32),
            pltpu.VMEM((NWN, BW), jnp.int32),
            pltpu.VMEM((NCH * 2, CTX_PER_CH // 2), jnp.int32),
            pltpu.VMEM((NCH, WN_PER_CH), jnp.int32),
            pltpu.VMEM((BW,), jnp.float32),
            pltpu.VMEM((NEG, BW), jnp.float32),
            pltpu.VMEM((CTX_PER_CH, 2 * D), jnp.int32),
            pltpu.VMEM((CTX_PER_CH, 2 * D), jnp.int32),
            pltpu.VMEM((CTX_PER_CH, 2 * D), jnp.int32),
            pltpu.VMEM((WN_PER_CH, 2 * D), jnp.int32),
            pltpu.VMEM((WN_PER_CH, 2 * D), jnp.int32),
            pltpu.VMEM((WN_PER_CH, 2 * D), jnp.int32),
            pltpu.VMEM((16,), jnp.float32),
            pltpu.SemaphoreType.DMA,
            pltpu.SemaphoreType.DMA,
            pltpu.SemaphoreType.DMA,
        ],
    )
    partials = run(emb0_r, emb1_r, ctx_t, wn_t, lens_r, mask_r)
    return jnp.sum(partials)
